# baseline (speedup 1.0000x reference)
import jax
import jax.numpy as jnp
from jax.experimental import pallas as pl

_EMBED_DIM = 128
_NUM_FILT = 200
_KER = 7
_KW = 8
_KH = 16
_FLAT_SZ = (2 * _KW - _KER + 1) * (_KH - _KER + 1) * _NUM_FILT


def _bn(x, axes):
    m = jnp.mean(x, axis=axes, keepdims=True)
    v = jnp.var(x, axis=axes, keepdims=True)
    return (x - m) / jnp.sqrt(v + 1e-5)


def _node_update_kern(agg_ref, x_ref, w_ref, ws_ref, out_ref):
    out_ref[...] = jnp.tanh(
        jnp.dot(agg_ref[...], w_ref[...], preferred_element_type=jnp.float32)
        + jnp.dot(x_ref[...], ws_ref[...], preferred_element_type=jnp.float32)
    )


def _node_update(agg, x, w, w_self):
    n, d = x.shape
    tm = 2000
    return pl.pallas_call(
        _node_update_kern,
        grid=(n // tm,),
        in_specs=[
            pl.BlockSpec((tm, d), lambda i: (i, 0)),
            pl.BlockSpec((tm, d), lambda i: (i, 0)),
            pl.BlockSpec((d, d), lambda i: (0, 0)),
            pl.BlockSpec((d, d), lambda i: (0, 0)),
        ],
        out_specs=pl.BlockSpec((tm, d), lambda i: (i, 0)),
        out_shape=jax.ShapeDtypeStruct((n, d), jnp.float32),
    )(agg, x, w, w_self)


def _fc_kern(h_ref, w_ref, out_ref):
    @pl.when(pl.program_id(0) == 0)
    def _():
        out_ref[...] = jnp.zeros_like(out_ref)

    out_ref[...] += jnp.dot(
        h_ref[...], w_ref[...], preferred_element_type=jnp.float32
    )


def _fc(h, w):
    b, k = h.shape
    kp = 20480
    h = jnp.pad(h, ((0, 0), (0, kp - k)))
    w = jnp.pad(w, ((0, kp - k), (0, 0)))
    tk = 2560
    d = w.shape[1]
    return pl.pallas_call(
        _fc_kern,
        grid=(kp // tk,),
        in_specs=[
            pl.BlockSpec((b, tk), lambda i: (0, i)),
            pl.BlockSpec((tk, d), lambda i: (i, 0)),
        ],
        out_specs=pl.BlockSpec((b, d), lambda i: (0, 0)),
        out_shape=jax.ShapeDtypeStruct((b, d), jnp.float32),
    )(h, w)


def _score_kern(h_ref, x_ref, bias_ref, out_ref):
    out_ref[...] = jax.nn.sigmoid(
        jnp.dot(h_ref[...], x_ref[...].T, preferred_element_type=jnp.float32)
        + bias_ref[...]
    )


def _score(h, x, ent_bias):
    b, d = h.shape
    n = x.shape[0]
    np_ = 10240
    x = jnp.pad(x, ((0, np_ - n), (0, 0)))
    bias = jnp.pad(ent_bias, (0, np_ - n)).reshape(1, np_)
    tn = 2048
    out = pl.pallas_call(
        _score_kern,
        grid=(np_ // tn,),
        in_specs=[
            pl.BlockSpec((b, d), lambda i: (0, 0)),
            pl.BlockSpec((tn, d), lambda i: (i, 0)),
            pl.BlockSpec((1, tn), lambda i: (0, i)),
        ],
        out_specs=pl.BlockSpec((b, tn), lambda i: (0, i)),
        out_shape=jax.ShapeDtypeStruct((b, np_), jnp.float32),
    )(h, x, bias)
    return out[:, :n]


def _ragcn_conv(x, src, dst, edge_type, rel_embed, w, w_self, w_rel, inv_deg):
    msg = x[src] - rel_embed[edge_type]
    agg = jax.ops.segment_sum(msg, dst, num_segments=x.shape[0])
    agg = agg * inv_deg[:, None]
    out = _node_update(agg, x, w, w_self)
    rel_out = rel_embed @ w_rel
    return out, rel_out


def kernel(init_embed, init_rel, W1, W1_self, W1_rel, W2, W2_self, W2_rel,
           conv_w, fc_w, fc_b, ent_bias, sub, rel, edge_index, edge_type):
    src = edge_index[0]
    dst = edge_index[1]
    num_ent = init_embed.shape[0]
    deg = jax.ops.segment_sum(
        jnp.ones((src.shape[0],), jnp.float32), dst, num_segments=num_ent
    )
    inv_deg = 1.0 / jnp.maximum(deg, 1.0)

    x, r = _ragcn_conv(init_embed, src, dst, edge_type, init_rel,
                       W1, W1_self, W1_rel, inv_deg)
    x, r = _ragcn_conv(x, src, dst, edge_type, r,
                       W2, W2_self, W2_rel, inv_deg)

    sub_emb = jnp.take(x, sub, axis=0)
    rel_emb = jnp.take(r, rel, axis=0)

    e1 = sub_emb.reshape(-1, 1, _EMBED_DIM)
    r2 = rel_emb.reshape(-1, 1, _EMBED_DIM)
    stk = jnp.concatenate([e1, r2], axis=1)
    stk = jnp.transpose(stk, (0, 2, 1)).reshape(-1, 1, 2 * _KW, _KH)
    h = _bn(stk, (0, 2, 3))
    h = jax.lax.conv_general_dilated(
        h, conv_w, (1, 1), 'VALID', dimension_numbers=('NCHW', 'OIHW', 'NCHW')
    )
    h = _bn(h, (0, 2, 3))
    h = jax.nn.relu(h)
    h = h.reshape(-1, _FLAT_SZ)
    h = _fc(h, fc_w) + fc_b
    h = _bn(h, (0,))
    h = jax.nn.relu(h)
    return _score(h, x, ent_bias)
